# Initial kernel scaffold; baseline (speedup 1.0000x reference)
import functools

import jax
import jax.numpy as jnp
from jax import lax
from jax.experimental import pallas as pl
from jax.experimental.pallas import tpu as pltpu
from jax.experimental.pallas import tpu_sc as plsc

N = 10000
E = 320000
G = 64
F_IN = 128
H = 32
N_MICRO = 30

NC = 2
NS = 16
NW = NC * NS
CH = 128
NCH = 79
EP = NW * NCH * CH
NP = 10016
RPT = NP // NS

_mesh = plsc.VectorSubcoreMesh(
    core_axis_name="c", subcore_axis_name="s", num_cores=NC, num_subcores=NS)


@functools.partial(
    pl.kernel,
    out_type=jax.ShapeDtypeStruct((NC, NP, 16), jnp.float32),
    mesh=_mesh,
    scratch_types=[
        pltpu.VMEM((NCH, CH), jnp.int32),
        pltpu.VMEM((CH, 16), jnp.float32),
        pltpu.VMEM_SHARED((NP, 16), jnp.float32),
    ],
)
def _sc_degree(dst_hbm, ones_hbm, z16_hbm, out_hbm, dst_v, ones_v, acc_sh):
    c = lax.axis_index("c")
    s = lax.axis_index("s")
    wid = c * NS + s
    pltpu.sync_copy(z16_hbm.at[pl.ds(s * RPT, RPT)], acc_sh.at[pl.ds(s * RPT, RPT)])
    pltpu.sync_copy(dst_hbm.at[wid], dst_v)
    pltpu.sync_copy(ones_hbm, ones_v)
    plsc.subcore_barrier()

    def body(j, carry):
        pltpu.sync_copy(ones_v, acc_sh.at[dst_v.at[j]], add=True)
        return carry

    lax.fori_loop(0, NCH, body, 0)
    plsc.subcore_barrier()
    pltpu.sync_copy(acc_sh.at[pl.ds(s * RPT, RPT)],
                    out_hbm.at[c, pl.ds(s * RPT, RPT)])


@functools.partial(
    pl.kernel,
    out_type=jax.ShapeDtypeStruct((NC, NP, H), jnp.float32),
    mesh=_mesh,
    scratch_types=[
        pltpu.VMEM((NCH, CH), jnp.int32),
        pltpu.VMEM((NCH, CH), jnp.int32),
        pltpu.VMEM((CH, H), jnp.float32),
        pltpu.VMEM_SHARED((NP, H), jnp.float32),
    ],
)
def _sc_agg(hp_hbm, src_hbm, dst_hbm, z_hbm, out_hbm, src_v, dst_v, rows_v, acc_sh):
    c = lax.axis_index("c")
    s = lax.axis_index("s")
    wid = c * NS + s
    pltpu.sync_copy(z_hbm.at[pl.ds(s * RPT, RPT)], acc_sh.at[pl.ds(s * RPT, RPT)])
    pltpu.sync_copy(src_hbm.at[wid], src_v)
    pltpu.sync_copy(dst_hbm.at[wid], dst_v)
    plsc.subcore_barrier()

    def body(j, carry):
        pltpu.sync_copy(hp_hbm.at[src_v.at[j]], rows_v)
        pltpu.sync_copy(rows_v, acc_sh.at[dst_v.at[j]], add=True)
        return carry

    lax.fori_loop(0, NCH, body, 0)
    plsc.subcore_barrier()
    pltpu.sync_copy(acc_sh.at[pl.ds(s * RPT, RPT)],
                    out_hbm.at[c, pl.ds(s * RPT, RPT)])


def _tc_pre_body(x_ref, w1_ref, degp_ref, hp_ref, dinv_ref):
    deg = degp_ref[0, :, 0:1] + degp_ref[1, :, 0:1] + 1.0
    dinv = lax.rsqrt(deg)
    h = jnp.dot(x_ref[...], w1_ref[...], preferred_element_type=jnp.float32)
    hp_ref[...] = dinv * h
    dinv_ref[...] = dinv


def _tc_mid_body(aggp_ref, hp_ref, dinv_ref, b_ref, w_ref, out_ref):
    dinv = dinv_ref[...]
    pre = dinv * (aggp_ref[0] + aggp_ref[1] + hp_ref[...]) + b_ref[...]
    a = jnp.maximum(pre, 0.0)
    out_ref[...] = dinv * jnp.dot(a, w_ref[...], preferred_element_type=jnp.float32)


def _tc_head_body(aggp_ref, hp_ref, dinv_ref, b_ref, batch_ref,
                  fw1_ref, fb1_ref, fw2_ref, fb2_ref, out_ref):
    dinv = dinv_ref[...]
    pre = dinv * (aggp_ref[0] + aggp_ref[1] + hp_ref[...]) + b_ref[...]
    a = jnp.maximum(pre, 0.0)[:N, :]
    gid = lax.broadcasted_iota(jnp.int32, (G, N), 0)
    oh = (gid == batch_ref[...]).astype(jnp.float32)
    sums = jnp.dot(oh, a, preferred_element_type=jnp.float32)
    cnt = jnp.sum(oh, axis=1, keepdims=True)
    pooled = sums / jnp.maximum(cnt, 1.0)
    z = jnp.maximum(
        jnp.dot(pooled, fw1_ref[...], preferred_element_type=jnp.float32)
        + fb1_ref[...], 0.0)
    z2 = jnp.dot(z, fw2_ref[...], preferred_element_type=jnp.float32) + fb2_ref[...]
    m = jnp.max(z2, axis=1, keepdims=True)
    e = jnp.exp(z2 - m)
    out_ref[...] = e / jnp.sum(e, axis=1, keepdims=True)


_tc_pre = pl.pallas_call(
    _tc_pre_body,
    out_shape=(jax.ShapeDtypeStruct((NP, H), jnp.float32),
               jax.ShapeDtypeStruct((NP, 1), jnp.float32)),
)

_tc_mid = pl.pallas_call(
    _tc_mid_body,
    out_shape=jax.ShapeDtypeStruct((NP, H), jnp.float32),
)

_tc_head = pl.pallas_call(
    _tc_head_body,
    out_shape=jax.ShapeDtypeStruct((G, N_MICRO), jnp.float32),
)


def kernel(x, edge_index, batch, W1, b1, W2, b2, W3, b3, W4, b4,
           fW1, fb1, fW2, fb2):
    src = edge_index[0]
    dst = edge_index[1]
    pad = EP - E
    src3 = jnp.concatenate([src, jnp.zeros((pad,), jnp.int32)]).reshape(NW, NCH, CH)
    dst3 = jnp.concatenate([dst, jnp.full((pad,), N, jnp.int32)]).reshape(NW, NCH, CH)

    ones16 = jnp.ones((CH, 16), jnp.float32)
    z16 = jnp.zeros((NP, 16), jnp.float32)
    zH = jnp.zeros((NP, H), jnp.float32)
    x_pad = jnp.concatenate([x, jnp.zeros((NP - N, F_IN), x.dtype)], axis=0)
    batch2 = batch.reshape(1, N)

    degp = _sc_degree(dst3, ones16, z16)
    hp, dinv = _tc_pre(x_pad, W1, degp)

    aggp = _sc_agg(hp, src3, dst3, zH)
    hp = _tc_mid(aggp, hp, dinv, b1.reshape(1, H), W2)

    aggp = _sc_agg(hp, src3, dst3, zH)
    hp = _tc_mid(aggp, hp, dinv, b2.reshape(1, H), W3)

    aggp = _sc_agg(hp, src3, dst3, zH)
    hp = _tc_mid(aggp, hp, dinv, b3.reshape(1, H), W4)

    aggp = _sc_agg(hp, src3, dst3, zH)
    return _tc_head(aggp, hp, dinv, b4.reshape(1, H), batch2,
                    fW1, fb1.reshape(1, 64), fW2, fb2.reshape(1, N_MICRO))


# speedup vs baseline: 22.4934x; 22.4934x over previous
import functools

import jax
import jax.numpy as jnp
from jax import lax
from jax.experimental import pallas as pl
from jax.experimental.pallas import tpu as pltpu
from jax.experimental.pallas import tpu_sc as plsc

N = 10000
E = 320000
G = 64
F_IN = 128
H = 32
N_MICRO = 30

NC = 2
NS = 16
NW = NC * NS
CH = 128
NCH = 79
EP = NW * NCH * CH
NP = 10112
RPT = NP // NS

_mesh = plsc.VectorSubcoreMesh(
    core_axis_name="c", subcore_axis_name="s", num_cores=NC, num_subcores=NS)
_sc_params = pltpu.CompilerParams(use_tc_tiling_on_sc=False)


@functools.partial(
    pl.kernel,
    out_type=jax.ShapeDtypeStruct((NC, NP, 16), jnp.float32),
    mesh=_mesh,
    scratch_types=[
        pltpu.VMEM((NCH, CH), jnp.int32),
        pltpu.VMEM((CH, 16), jnp.float32),
        pltpu.VMEM_SHARED((NP, 16), jnp.float32),
    ],
    compiler_params=_sc_params,
)
def _sc_degree(dst_hbm, ones_hbm, z16_hbm, out_hbm, dst_v, ones_v, acc_sh):
    c = lax.axis_index("c")
    s = lax.axis_index("s")
    wid = c * NS + s
    pltpu.sync_copy(z16_hbm.at[pl.ds(s * RPT, RPT)], acc_sh.at[pl.ds(s * RPT, RPT)])
    pltpu.sync_copy(dst_hbm.at[wid], dst_v)
    pltpu.sync_copy(ones_hbm, ones_v)
    plsc.subcore_barrier()

    def body(j, carry):
        pltpu.sync_copy(ones_v, acc_sh.at[dst_v.at[j]], add=True)
        return carry

    lax.fori_loop(0, NCH, body, 0)
    plsc.subcore_barrier()
    pltpu.sync_copy(acc_sh.at[pl.ds(s * RPT, RPT)],
                    out_hbm.at[c, pl.ds(s * RPT, RPT)])


@functools.partial(
    pl.kernel,
    out_type=jax.ShapeDtypeStruct((NC, NP, H), jnp.float32),
    mesh=_mesh,
    scratch_types=[
        pltpu.VMEM((NCH, CH), jnp.int32),
        pltpu.VMEM((NCH, CH), jnp.int32),
        pltpu.VMEM((CH, H), jnp.float32),
        pltpu.VMEM_SHARED((NP, H), jnp.float32),
    ],
    compiler_params=_sc_params,
)
def _sc_agg(hp_hbm, src_hbm, dst_hbm, z_hbm, out_hbm, src_v, dst_v, rows_v, acc_sh):
    c = lax.axis_index("c")
    s = lax.axis_index("s")
    wid = c * NS + s
    pltpu.sync_copy(z_hbm.at[pl.ds(s * RPT, RPT)], acc_sh.at[pl.ds(s * RPT, RPT)])
    pltpu.sync_copy(src_hbm.at[wid], src_v)
    pltpu.sync_copy(dst_hbm.at[wid], dst_v)
    plsc.subcore_barrier()

    def body(j, carry):
        pltpu.sync_copy(hp_hbm.at[src_v.at[j]], rows_v)
        pltpu.sync_copy(rows_v, acc_sh.at[dst_v.at[j]], add=True)
        return carry

    lax.fori_loop(0, NCH, body, 0)
    plsc.subcore_barrier()
    pltpu.sync_copy(acc_sh.at[pl.ds(s * RPT, RPT)],
                    out_hbm.at[c, pl.ds(s * RPT, RPT)])


def _tc_pre_body(x_ref, w1_ref, degp_ref, hp_ref, dinv_ref):
    deg = degp_ref[0, :, 0:1] + degp_ref[1, :, 0:1] + 1.0
    dinv = lax.rsqrt(deg)
    h = jnp.dot(x_ref[...], w1_ref[...], preferred_element_type=jnp.float32)
    hp_ref[...] = dinv * h
    dinv_ref[...] = dinv


def _tc_mid_body(aggp_ref, hp_ref, dinv_ref, b_ref, w_ref, out_ref):
    dinv = dinv_ref[...]
    pre = dinv * (aggp_ref[0] + aggp_ref[1] + hp_ref[...]) + b_ref[...]
    a = jnp.maximum(pre, 0.0)
    out_ref[...] = dinv * jnp.dot(a, w_ref[...], preferred_element_type=jnp.float32)


def _tc_head_body(aggp_ref, hp_ref, dinv_ref, b_ref, batch_ref,
                  fw1_ref, fb1_ref, fw2_ref, fb2_ref, out_ref):
    dinv = dinv_ref[...]
    pre = dinv * (aggp_ref[0] + aggp_ref[1] + hp_ref[...]) + b_ref[...]
    a = jnp.maximum(pre, 0.0)[:N, :]
    gid = lax.broadcasted_iota(jnp.int32, (G, N), 0)
    oh = (gid == batch_ref[...]).astype(jnp.float32)
    sums = jnp.dot(oh, a, preferred_element_type=jnp.float32)
    cnt = jnp.sum(oh, axis=1, keepdims=True)
    pooled = sums / jnp.maximum(cnt, 1.0)
    z = jnp.maximum(
        jnp.dot(pooled, fw1_ref[...], preferred_element_type=jnp.float32)
        + fb1_ref[...], 0.0)
    z2 = jnp.dot(z, fw2_ref[...], preferred_element_type=jnp.float32) + fb2_ref[...]
    m = jnp.max(z2, axis=1, keepdims=True)
    e = jnp.exp(z2 - m)
    out_ref[...] = e / jnp.sum(e, axis=1, keepdims=True)


_tc_pre = pl.pallas_call(
    _tc_pre_body,
    out_shape=(jax.ShapeDtypeStruct((NP, H), jnp.float32),
               jax.ShapeDtypeStruct((NP, 1), jnp.float32)),
)

_tc_mid = pl.pallas_call(
    _tc_mid_body,
    out_shape=jax.ShapeDtypeStruct((NP, H), jnp.float32),
)

_tc_head = pl.pallas_call(
    _tc_head_body,
    out_shape=jax.ShapeDtypeStruct((G, N_MICRO), jnp.float32),
)


def kernel(x, edge_index, batch, W1, b1, W2, b2, W3, b3, W4, b4,
           fW1, fb1, fW2, fb2):
    src = edge_index[0]
    dst = edge_index[1]
    pad = EP - E
    src3 = jnp.concatenate([src, jnp.zeros((pad,), jnp.int32)]).reshape(NW, NCH, CH)
    dst3 = jnp.concatenate([dst, jnp.full((pad,), N, jnp.int32)]).reshape(NW, NCH, CH)

    ones16 = jnp.ones((CH, 16), jnp.float32)
    z16 = jnp.zeros((NP, 16), jnp.float32)
    zH = jnp.zeros((NP, H), jnp.float32)
    x_pad = jnp.concatenate([x, jnp.zeros((NP - N, F_IN), x.dtype)], axis=0)
    batch2 = batch.reshape(1, N)

    degp = _sc_degree(dst3, ones16, z16)
    hp, dinv = _tc_pre(x_pad, W1, degp)

    aggp = _sc_agg(hp, src3, dst3, zH)
    hp = _tc_mid(aggp, hp, dinv, b1.reshape(1, H), W2)

    aggp = _sc_agg(hp, src3, dst3, zH)
    hp = _tc_mid(aggp, hp, dinv, b2.reshape(1, H), W3)

    aggp = _sc_agg(hp, src3, dst3, zH)
    hp = _tc_mid(aggp, hp, dinv, b3.reshape(1, H), W4)

    aggp = _sc_agg(hp, src3, dst3, zH)
    return _tc_head(aggp, hp, dinv, b4.reshape(1, H), batch2,
                    fW1, fb1.reshape(1, 64), fW2, fb2.reshape(1, N_MICRO))
